# baseline (speedup 1.0000x reference)
import jax
import jax.numpy as jnp
import numpy as np
from functools import partial
from jax.experimental import pallas as pl
from jax.experimental.pallas import tpu as pltpu

_NCLS = 19
_B = 4
_C = 256
_CT = 32
_HW = 16384
_NBIG = 175
_NSMALL = 75
_CHUNK = 2048
_NCH = _HW // _CHUNK
_TINY = np.float32(np.finfo(np.float32).tiny)


def _interp_kernel(x_ref, rm_ref, cm_ref, xi_ref, ssq_ref):
    ct = pl.program_id(1)
    xb = x_ref[0]
    tmp = jax.lax.dot_general(xb, cm_ref[...], (((2,), (1,)), ((), ())),
                              preferred_element_type=jnp.float32)
    rmb = jnp.broadcast_to(rm_ref[...][None], (_CT, 128, 64))
    xi = jax.lax.dot_general(rmb, tmp, (((2,), (1,)), ((0,), (0,))),
                             preferred_element_type=jnp.float32)
    xi_ref[0] = xi
    psum = jnp.sum(xi * xi, axis=0)

    @pl.when(ct == 0)
    def _():
        ssq_ref[0] = psum

    @pl.when(ct != 0)
    def _():
        ssq_ref[0] += psum


def _group_mask_int(pm_ref, lb_ref, j, off, gbase):
    cls = j // (2 * _B)
    gg = j % 2
    pmc = pm_ref[0, :, pl.ds(off, _CHUNK)]
    lbc = lb_ref[0, :, pl.ds(off, _CHUNK)]
    peq = pmc == cls
    leq = lbc == cls
    if gbase == 0:
        m0 = jnp.logical_and(peq, leq)
        m1 = jnp.logical_and(~peq, ~leq)
    else:
        m0 = jnp.logical_and(peq, ~leq)
        m1 = jnp.logical_and(~peq, leq)
    return jnp.where(gg == 0, m0.astype(jnp.int32), m1.astype(jnp.int32))


def _kmax_kernel(bits_ref, pm_ref, lb_ref, kmm_ref, kma_ref,
                 mm_ref, ma_ref, *, count, gbase):
    j = pl.program_id(0)
    ch = pl.program_id(1)
    off = ch * _CHUNK
    mi = _group_mask_int(pm_ref, lb_ref, j, off, gbase)
    k = jax.lax.shift_right_logical(bits_ref[0],
                                    jnp.uint32(9)).astype(jnp.int32)
    km = jnp.where(mi > 0, k, -1)
    cmm = jnp.max(km, axis=1, keepdims=True)
    cma = jnp.max(k, axis=1, keepdims=True)

    @pl.when(ch == 0)
    def _():
        mm_ref[...] = cmm
        ma_ref[...] = cma

    @pl.when(ch != 0)
    def _():
        mm_ref[...] = jnp.maximum(mm_ref[...], cmm)
        ma_ref[...] = jnp.maximum(ma_ref[...], cma)

    @pl.when(ch == _NCH - 1)
    def _():
        kmm_ref[0] = mm_ref[...]
        kma_ref[0] = ma_ref[...]


def _select_kernel(bits_ref, pm_ref, lb_ref, kthr_ref, emp_ref,
                   c1_ref, c2_ref, c3_ref, c4_ref,
                   r1_ref, r2_ref, r3_ref, r4_ref, *, count, gbase):
    j = pl.program_id(0)
    ch = pl.program_id(1)
    off = ch * _CHUNK
    mi = _group_mask_int(pm_ref, lb_ref, j, off, gbase)
    k = jax.lax.shift_right_logical(bits_ref[0],
                                    jnp.uint32(9)).astype(jnp.int32)
    ok = jnp.logical_and(
        jnp.logical_or(mi > 0, emp_ref[0] > 0),
        k >= kthr_ref[0])
    iota = jax.lax.broadcasted_iota(jnp.int32, (count, _CHUNK), 1) + off
    idx = jnp.where(ok, iota, _HW)
    m1 = jnp.min(idx, axis=1, keepdims=True)
    idx = jnp.where(idx == m1, _HW, idx)
    m2 = jnp.min(idx, axis=1, keepdims=True)
    idx = jnp.where(idx == m2, _HW, idx)
    m3 = jnp.min(idx, axis=1, keepdims=True)
    idx = jnp.where(idx == m3, _HW, idx)
    m4 = jnp.min(idx, axis=1, keepdims=True)

    @pl.when(ch == 0)
    def _():
        r1_ref[...] = m1
        r2_ref[...] = m2
        r3_ref[...] = m3
        r4_ref[...] = m4

    @pl.when(ch != 0)
    def _():
        rv1, rv2 = r1_ref[...], r2_ref[...]
        rv3, rv4 = r3_ref[...], r4_ref[...]
        v1, v2, v3 = rv1 < _HW, rv2 < _HW, rv3 < _HW
        r2_ref[...] = jnp.where(v2, rv2, jnp.where(v1, m1, m2))
        r3_ref[...] = jnp.where(v3, rv3,
                                jnp.where(v2, m1, jnp.where(v1, m2, m3)))
        r4_ref[...] = jnp.where(rv4 < _HW, rv4,
                                jnp.where(v3, m1,
                                          jnp.where(v2, m2,
                                                    jnp.where(v1, m3, m4))))
        r1_ref[...] = jnp.where(v1, rv1, m1)

    @pl.when(ch == _NCH - 1)
    def _():
        c1_ref[0] = r1_ref[...]
        c2_ref[0] = r2_ref[...]
        c3_ref[0] = r3_ref[...]
        c4_ref[0] = r4_ref[...]


def _hist_kernel(win_ref, ssq_ref, hist_ref, *, count):
    winners = win_ref[0]
    for cc in range(_NCH):
        it2 = jax.lax.broadcasted_iota(jnp.int32, (count, _CHUNK), 1)
        it2 = it2 + cc * _CHUNK
        cnt = jnp.sum(jnp.where(winners == it2, 1.0, 0.0), axis=0,
                      keepdims=True)
        w = 1.0 / (1e-6 + jnp.sqrt(ssq_ref[0, :, cc * _CHUNK:
                                           (cc + 1) * _CHUNK]))
        hist_ref[0, :, cc * _CHUNK:(cc + 1) * _CHUNK] = cnt * w


def _contract_kernel(xi_ref, hb_ref, hs_ref, out_ref, acc_ref):
    b = pl.program_id(0)
    ch = pl.program_id(1)
    xb = xi_ref[0]
    hb = hb_ref[:, 0, :, :]
    hs = hs_ref[:, 0, :, :]
    rhs = jnp.concatenate(
        [hb[:, 0, :], hb[:, 1, :], hs[:, 0, :], hs[:, 1, :]], axis=0)
    part = jax.lax.dot_general(xb, rhs, (((1,), (1,)), ((), ())),
                               preferred_element_type=jnp.float32)

    @pl.when(jnp.logical_and(b == 0, ch == 0))
    def _():
        acc_ref[...] = part

    @pl.when(jnp.logical_not(jnp.logical_and(b == 0, ch == 0)))
    def _():
        acc_ref[...] += part

    @pl.when(jnp.logical_and(b == _B - 1, ch == _NCH - 1))
    def _():
        s = acc_ref[...]
        d1 = s[:, 57:76] - s[:, 38:57]
        d2 = s[:, 0:19] - s[:, 19:38]
        tot = jnp.sum(d1 * d2, axis=(0, 1), keepdims=True)
        out_ref[...] = tot / (1200.0 * 700.0) / float(_NCLS)


def _gumbel_val(kv, empty):
    f = kv.astype(jnp.float32) * np.float32(2.0 ** -23)
    u = jnp.maximum(f + _TINY, _TINY)
    g = -jnp.log(-jnp.log(u))
    return jnp.where(empty, g + jnp.float32(-1e9), g)


def _thresholds(kmm, kma):
    kmm = kmm[..., 0]
    kma = kma[..., 0]
    empty = kmm < 0
    k_eff = jnp.where(empty, kma, kmm)
    tgt = _gumbel_val(k_eff, empty)
    for _ in range(128):
        tgt = jnp.nextafter(tgt, jnp.float32(-jnp.inf))
    lo = jnp.zeros_like(k_eff)
    hi = k_eff
    for _ in range(24):
        mid = (lo + hi) // 2
        ge = _gumbel_val(mid, empty) >= tgt
        hi = jnp.where(ge, mid, hi)
        lo = jnp.where(ge, lo, mid + 1)
    return hi[..., None], empty[..., None].astype(jnp.int32)


def kernel(x, logit, label):
    x = x.astype(jnp.float32)
    logit = logit.astype(jnp.float32)

    rows = jnp.linspace(0.0, 63.0, 128)
    r0 = jnp.floor(rows).astype(jnp.int32)
    r1 = jnp.minimum(r0 + 1, 63)
    fr = (rows - r0.astype(jnp.float32)).astype(jnp.float32)
    eye = jnp.eye(64, dtype=jnp.float32)
    rmat = eye[r0] * (1.0 - fr)[:, None] + eye[r1] * fr[:, None]

    pred = jax.nn.softmax(logit, axis=1)
    pm = jnp.argmax(jnp.exp(pred), axis=1).astype(jnp.int32)
    pm = pm.reshape(_B, 1, _HW)

    skey = jax.random.key(42)
    folds_big, folds_small = [], []
    for cls in range(_NCLS):
        for bi in range(_B):
            base = cls * 100 + bi * 10
            folds_big += [base + 0, base + 1]
            folds_small += [base + 2, base + 3]
    kb = jax.vmap(jax.random.fold_in, (None, 0))(skey, jnp.array(folds_big))
    ks = jax.vmap(jax.random.fold_in, (None, 0))(skey, jnp.array(folds_small))
    bits_big = jax.vmap(
        lambda k: jax.random.bits(k, (_NBIG, _HW), jnp.uint32))(kb)
    bits_small = jax.vmap(
        lambda k: jax.random.bits(k, (_NSMALL, _HW), jnp.uint32))(ks)

    xi, ssq = pl.pallas_call(
        _interp_kernel,
        grid=(_B, _C // _CT),
        in_specs=[
            pl.BlockSpec((1, _CT, 64, 64), lambda b, t: (b, t, 0, 0)),
            pl.BlockSpec((128, 64), lambda b, t: (0, 0)),
            pl.BlockSpec((128, 64), lambda b, t: (0, 0)),
        ],
        out_specs=[
            pl.BlockSpec((1, _CT, 128, 128), lambda b, t: (b, t, 0, 0)),
            pl.BlockSpec((1, 128, 128), lambda b, t: (b, 0, 0)),
        ],
        out_shape=[
            jax.ShapeDtypeStruct((_B, _C, 128, 128), jnp.float32),
            jax.ShapeDtypeStruct((_B, 128, 128), jnp.float32),
        ],
    )(x, rmat, rmat)

    lb = label.astype(jnp.int32).reshape(_B, 1, _HW)
    ssq3 = ssq.reshape(_B, 1, _HW)
    ngroups = 2 * _B * _NCLS

    def run_sampler(bits, count, gbase):
        row_spec = pl.BlockSpec((1, 1, _HW), lambda j, ch: ((j // 2) % _B, 0, 0))
        kmm, kma = pl.pallas_call(
            partial(_kmax_kernel, count=count, gbase=gbase),
            grid=(ngroups, _NCH),
            in_specs=[
                pl.BlockSpec((1, count, _CHUNK), lambda j, ch: (j, 0, ch)),
                row_spec, row_spec,
            ],
            out_specs=[
                pl.BlockSpec((1, count, 1), lambda j, ch: (j, 0, 0)),
                pl.BlockSpec((1, count, 1), lambda j, ch: (j, 0, 0)),
            ],
            out_shape=[
                jax.ShapeDtypeStruct((ngroups, count, 1), jnp.int32),
                jax.ShapeDtypeStruct((ngroups, count, 1), jnp.int32),
            ],
            scratch_shapes=[
                pltpu.VMEM((count, 1), jnp.int32),
                pltpu.VMEM((count, 1), jnp.int32),
            ],
        )(bits, pm, lb)
        kthr, emp = _thresholds(kmm, kma)
        cand_spec = pl.BlockSpec((1, count, 1), lambda j, ch: (j, 0, 0))
        cands = pl.pallas_call(
            partial(_select_kernel, count=count, gbase=gbase),
            grid=(ngroups, _NCH),
            in_specs=[
                pl.BlockSpec((1, count, _CHUNK), lambda j, ch: (j, 0, ch)),
                row_spec, row_spec, cand_spec, cand_spec,
            ],
            out_specs=[cand_spec] * 4,
            out_shape=[
                jax.ShapeDtypeStruct((ngroups, count, 1), jnp.int32)] * 4,
            scratch_shapes=[pltpu.VMEM((count, 1), jnp.int32)] * 4,
        )(bits, pm, lb, kthr, emp)

        empty = emp > 0
        winners = cands[0]
        bw = jnp.take_along_axis(bits, winners.astype(jnp.uint32), axis=2)
        bestv = _gumbel_val((bw >> 9).astype(jnp.int32), empty)
        for cn in cands[1:]:
            bn = jnp.take_along_axis(bits, cn.astype(jnp.uint32) % _HW,
                                     axis=2)
            vn = _gumbel_val((bn >> 9).astype(jnp.int32), empty)
            take = jnp.logical_and(cn < _HW, vn > bestv)
            winners = jnp.where(take, cn, winners)
            bestv = jnp.where(take, vn, bestv)

        return pl.pallas_call(
            partial(_hist_kernel, count=count),
            grid=(ngroups,),
            in_specs=[
                pl.BlockSpec((1, count, 1), lambda j: (j, 0, 0)),
                pl.BlockSpec((1, 1, _HW), lambda j: ((j // 2) % _B, 0, 0)),
            ],
            out_specs=pl.BlockSpec((1, 1, _HW), lambda j: (j, 0, 0)),
            out_shape=jax.ShapeDtypeStruct((ngroups, 1, _HW), jnp.float32),
        )(winners, ssq3)

    hist_big = run_sampler(bits_big, _NBIG, 0)
    hist_small = run_sampler(bits_small, _NSMALL, 2)

    hb4 = hist_big.reshape(_NCLS, _B, 2, _HW)
    hs4 = hist_small.reshape(_NCLS, _B, 2, _HW)

    out = pl.pallas_call(
        _contract_kernel,
        grid=(_B, _NCH),
        in_specs=[
            pl.BlockSpec((1, _C, _CHUNK), lambda b, ch: (b, 0, ch)),
            pl.BlockSpec((_NCLS, 1, 2, _CHUNK), lambda b, ch: (0, b, 0, ch)),
            pl.BlockSpec((_NCLS, 1, 2, _CHUNK), lambda b, ch: (0, b, 0, ch)),
        ],
        out_specs=pl.BlockSpec((1, 1), lambda b, ch: (0, 0)),
        out_shape=jax.ShapeDtypeStruct((1, 1), jnp.float32),
        scratch_shapes=[pltpu.VMEM((_C, 76), jnp.float32)],
    )(xi.reshape(_B, _C, _HW), hb4, hs4)

    return out[0, 0]
